# baseline (speedup 1.0000x reference)
import jax
import jax.numpy as jnp
from jax import lax
from jax.experimental import pallas as pl
from jax.experimental.pallas import tpu as pltpu
from jax.experimental.pallas import tpu_sc as plsc

_NC, _NS = 2, 16
_E = 320000
_D = 128
_N = 10000
_HALF = _N // _NC
_EPT = _E // _NS
_B = 128
_NFULL = _EPT // _B
_TAIL = _EPT - _NFULL * _B
_NBLK = _NFULL + 1
_NG = _NFULL // 3
_ACC_ROWS = 5120
_ZPT = _ACC_ROWS // _NS
_OPT = 312


def _sc_body(msg_hbm, idx_hbm, out_hbm, b0, b1, b2, idx_v, zbuf, acc,
             l0, l1, l2, c0, c1, c2):
    c = lax.axis_index("c")
    s = lax.axis_index("s")
    row0 = s * _EPT
    bufs = (b0, b1, b2)
    lsem = (l0, l1, l2)
    csem = (c0, c1, c2)

    def load(j, k):
        pltpu.async_copy(msg_hbm.at[pl.ds(row0 + j * _B, _B)], bufs[k], lsem[k])

    def load_wait(j, k):
        pltpu.make_async_copy(msg_hbm.at[pl.ds(row0 + j * _B, _B)],
                              bufs[k], lsem[k]).wait()

    def scat(j, k):
        pltpu.async_copy(bufs[k], acc.at[idx_v.at[j]], csem[k], add=True)

    def scat_wait(j, k):
        pltpu.make_async_copy(bufs[k], acc.at[idx_v.at[j]], csem[k]).wait()

    load(0, 0)
    load(1, 1)

    pltpu.sync_copy(idx_hbm.at[c, s], idx_v)

    def _z(r, carry):
        for k in range(_D // 16):
            zbuf[r, pl.ds(k * 16, 16)] = jnp.zeros((16,), jnp.float32)
        return carry

    lax.fori_loop(0, 128, _z, 0)
    pltpu.sync_copy(zbuf, acc.at[pl.ds(s * _ZPT, 128)])
    pltpu.sync_copy(zbuf, acc.at[pl.ds(s * _ZPT + 128, 128)])
    pltpu.sync_copy(zbuf.at[pl.ds(0, 64)], acc.at[pl.ds(s * _ZPT + 256, 64)])
    plsc.subcore_barrier()

    def _group(i, carry):
        for k in range(3):
            j = i * 3 + k
            k2 = (k + 2) % 3
            load_wait(j, k)
            scat(j, k)
            if k == 0:

                @pl.when(i > 0)
                def _():
                    scat_wait(j - 1, k2)

                load(j + 2, k2)
            else:
                scat_wait(j - 1, k2)

                @pl.when(i < _NG - 1)
                def _():
                    load(j + 2, k2)
        return carry

    lax.fori_loop(0, _NG, _group, 0)
    scat_wait(_NFULL - 1, (_NFULL - 1) % 3)

    pltpu.async_copy(msg_hbm.at[pl.ds(row0 + _NFULL * _B, _TAIL)],
                     b0.at[pl.ds(0, _TAIL)], l0)
    pltpu.make_async_copy(msg_hbm.at[pl.ds(row0 + _NFULL * _B, _TAIL)],
                          b0.at[pl.ds(0, _TAIL)], l0).wait()
    pltpu.sync_copy(b0, acc.at[idx_v.at[_NFULL]], add=True)

    plsc.subcore_barrier()

    pltpu.sync_copy(acc.at[pl.ds(s * _OPT, _OPT)],
                    out_hbm.at[pl.ds(c * _HALF + s * _OPT, _OPT)])

    @pl.when(s == _NS - 1)
    def _():
        pltpu.sync_copy(acc.at[pl.ds(_NS * _OPT, _HALF - _NS * _OPT)],
                        out_hbm.at[pl.ds(c * _HALF + _NS * _OPT,
                                         _HALF - _NS * _OPT)])


@jax.jit
def _run(messages, dst):
    dstp = jnp.concatenate(
        [dst.reshape(_NS, _EPT),
         jnp.full((_NS, _NBLK * _B - _EPT), jnp.int32(1 << 30))], axis=1)
    halves = []
    for core in range(_NC):
        local = dstp - jnp.int32(core * _HALF)
        ok = (local >= 0) & (local < _HALF)
        halves.append(jnp.where(ok, local, jnp.int32(_HALF)))
    idx = jnp.stack(halves).reshape(_NC, _NS, _NBLK, _B)

    mesh = plsc.VectorSubcoreMesh(core_axis_name="c", subcore_axis_name="s",
                                  num_cores=_NC, num_subcores=_NS)
    return pl.kernel(
        _sc_body,
        out_type=jax.ShapeDtypeStruct((_N, _D), jnp.float32),
        mesh=mesh,
        scratch_types=[
            pltpu.VMEM((_B, _D), jnp.float32),
            pltpu.VMEM((_B, _D), jnp.float32),
            pltpu.VMEM((_B, _D), jnp.float32),
            pltpu.VMEM((_NBLK, _B), jnp.int32),
            pltpu.VMEM((128, _D), jnp.float32),
            pltpu.VMEM_SHARED((_ACC_ROWS, _D), jnp.float32),
            pltpu.SemaphoreType.DMA,
            pltpu.SemaphoreType.DMA,
            pltpu.SemaphoreType.DMA,
            pltpu.SemaphoreType.DMA,
            pltpu.SemaphoreType.DMA,
            pltpu.SemaphoreType.DMA,
        ],
    )(messages, idx)


def kernel(messages, edge_index, num_nodes):
    return _run(messages, edge_index[1].astype(jnp.int32))


# speedup vs baseline: 1.3284x; 1.3284x over previous
import jax
import jax.numpy as jnp
from jax import lax
from jax.experimental import pallas as pl
from jax.experimental.pallas import tpu as pltpu
from jax.experimental.pallas import tpu_sc as plsc

_NC, _NS = 2, 16
_E = 320000
_D = 128
_N = 10000
_HALF = _N // _NC
_EPT = _E // _NS
_B = 128
_NFULL = _EPT // _B
_TAIL = _EPT - _NFULL * _B
_NBLK = _NFULL + 1
_NG = _NFULL // 3
_ACC_ROWS = 5120
_ZPT = _ACC_ROWS // _NS
_OPT = 312


def _sc_body(msg_hbm, idx_hbm, out_hbm, b0, b1, b2, idx_v, zbuf, acc,
             l0, l1, l2, c0, c1, c2):
    c = lax.axis_index("c")
    s = lax.axis_index("s")
    row0 = s * _EPT
    bufs = (b0, b1, b2)
    lsem = (l0, l1, l2)
    csem = (c0, c1, c2)

    def load(j, k):
        pltpu.async_copy(msg_hbm.at[pl.ds(row0 + j * _B, _B)], bufs[k], lsem[k])

    def load_wait(j, k):
        pltpu.make_async_copy(msg_hbm.at[pl.ds(row0 + j * _B, _B)],
                              bufs[k], lsem[k]).wait()

    def scat(j, k):
        pltpu.async_copy(bufs[k], acc.at[idx_v.at[j]], csem[k], add=True)

    def scat_wait(j, k):
        pltpu.make_async_copy(bufs[k], acc.at[idx_v.at[j]], csem[k]).wait()

    load(0, 0)
    load(1, 1)

    pltpu.sync_copy(idx_hbm.at[c, s], idx_v)

    def _z(r, carry):
        for k in range(_D // 16):
            zbuf[r, pl.ds(k * 16, 16)] = jnp.zeros((16,), jnp.float32)
        return carry

    lax.fori_loop(0, 128, _z, 0)
    pltpu.sync_copy(zbuf, acc.at[pl.ds(s * _ZPT, 128)])
    pltpu.sync_copy(zbuf, acc.at[pl.ds(s * _ZPT + 128, 128)])
    pltpu.sync_copy(zbuf.at[pl.ds(0, 64)], acc.at[pl.ds(s * _ZPT + 256, 64)])
    plsc.subcore_barrier()

    def _group(i, carry):
        for k in range(3):
            j = i * 3 + k
            k2 = (k + 2) % 3
            load_wait(j, k)
            scat(j, k)
            if k == 0:

                @pl.when(i > 0)
                def _():
                    scat_wait(j - 1, k2)

                load(j + 2, k2)
            else:
                scat_wait(j - 1, k2)

                @pl.when(i < _NG - 1)
                def _():
                    load(j + 2, k2)
        return carry

    lax.fori_loop(0, _NG, _group, 0)
    scat_wait(_NFULL - 1, (_NFULL - 1) % 3)

    pltpu.async_copy(msg_hbm.at[pl.ds(row0 + _NFULL * _B, _TAIL)],
                     b0.at[pl.ds(0, _TAIL)], l0)
    pltpu.make_async_copy(msg_hbm.at[pl.ds(row0 + _NFULL * _B, _TAIL)],
                          b0.at[pl.ds(0, _TAIL)], l0).wait()
    pltpu.sync_copy(b0, acc.at[idx_v.at[_NFULL]], add=True)

    plsc.subcore_barrier()

    pltpu.sync_copy(acc.at[pl.ds(s * _OPT, _OPT)],
                    out_hbm.at[pl.ds(c * _HALF + s * _OPT, _OPT)])

    @pl.when(s == _NS - 1)
    def _():
        pltpu.sync_copy(acc.at[pl.ds(_NS * _OPT, _HALF - _NS * _OPT)],
                        out_hbm.at[pl.ds(c * _HALF + _NS * _OPT,
                                         _HALF - _NS * _OPT)])


@jax.jit
def _run(messages, dst):
    dstp = jnp.concatenate(
        [dst.reshape(_NS, _EPT),
         jnp.full((_NS, _NBLK * _B - _EPT), jnp.int32(1 << 30))], axis=1)
    dummy = jnp.int32(_HALF) + (
        jax.lax.broadcasted_iota(jnp.int32, dstp.shape, 1)
        % jnp.int32(_ACC_ROWS - _HALF))
    halves = []
    for core in range(_NC):
        local = dstp - jnp.int32(core * _HALF)
        ok = (local >= 0) & (local < _HALF)
        halves.append(jnp.where(ok, local, dummy))
    idx = jnp.stack(halves).reshape(_NC, _NS, _NBLK, _B)

    mesh = plsc.VectorSubcoreMesh(core_axis_name="c", subcore_axis_name="s",
                                  num_cores=_NC, num_subcores=_NS)
    return pl.kernel(
        _sc_body,
        out_type=jax.ShapeDtypeStruct((_N, _D), jnp.float32),
        mesh=mesh,
        scratch_types=[
            pltpu.VMEM((_B, _D), jnp.float32),
            pltpu.VMEM((_B, _D), jnp.float32),
            pltpu.VMEM((_B, _D), jnp.float32),
            pltpu.VMEM((_NBLK, _B), jnp.int32),
            pltpu.VMEM((128, _D), jnp.float32),
            pltpu.VMEM_SHARED((_ACC_ROWS, _D), jnp.float32),
            pltpu.SemaphoreType.DMA,
            pltpu.SemaphoreType.DMA,
            pltpu.SemaphoreType.DMA,
            pltpu.SemaphoreType.DMA,
            pltpu.SemaphoreType.DMA,
            pltpu.SemaphoreType.DMA,
        ],
    )(messages, idx)


def kernel(messages, edge_index, num_nodes):
    return _run(messages, edge_index[1].astype(jnp.int32))
